# baseline (speedup 1.0000x reference)
import functools

import jax
import jax.numpy as jnp
from jax import lax
from jax.experimental import pallas as pl
from jax.experimental.pallas import tpu as pltpu
from jax.experimental.pallas import tpu_sc as plsc

NC = 2
NS = 16
L = 16
NW = NC * NS


def _rsqrt(x):
    i = plsc.bitcast(x, jnp.int32)
    i = jnp.int32(0x5F3759DF) - lax.shift_right_logical(i, 1)
    y = plsc.bitcast(i, jnp.float32)
    xh = x * jnp.float32(0.5)
    for _ in range(3):
        y = y * (jnp.float32(1.5) - xh * y * y)
    return y


def _make_sc_kernel(batch):
    b_per_w = batch // NW
    mesh = plsc.VectorSubcoreMesh(
        core_axis_name="c", subcore_axis_name="s", num_cores=NC, num_subcores=NS
    )

    @functools.partial(
        pl.kernel,
        mesh=mesh,
        out_type=jax.ShapeDtypeStruct((batch, 9), jnp.float32),
        scratch_types=[
            pltpu.VMEM((b_per_w,), jnp.int32),
            pltpu.VMEM((b_per_w, 3), jnp.float32),
            pltpu.VMEM((b_per_w, 9), jnp.float32),
            pltpu.SemaphoreType.DMA,
        ],
        compiler_params=pltpu.CompilerParams(
            needs_layout_passes=False, use_tc_tiling_on_sc=False
        ),
    )
    def sc_kernel(table_hbm, idx_hbm, out_hbm, idx_v, rows_v, out_v, sem):
        wid = lax.axis_index("s") * NC + lax.axis_index("c")
        base = wid * b_per_w
        pltpu.sync_copy(idx_hbm.at[pl.ds(base, b_per_w)], idx_v)
        pltpu.async_copy(table_hbm.at[idx_v], rows_v, sem).wait()

        lane = lax.iota(jnp.int32, L)
        c0 = jnp.zeros((L,), jnp.int32)
        c1 = jnp.full((L,), 1, jnp.int32)
        c2 = jnp.full((L,), 2, jnp.int32)
        zero_f = jnp.zeros((L,), jnp.float32)

        def body(g):
            rows16 = lane + g * L
            l0 = plsc.load_gather(rows_v, [rows16, c0])
            l1 = plsc.load_gather(rows_v, [rows16, c1])
            l2 = plsc.load_gather(rows_v, [rows16, c2])

            s = l0 * l0 + l1 * l1
            r2 = _rsqrt(s)
            r3 = _rsqrt(s + l2 * l2)
            q = r2 * r3
            t = l2 * q
            plsc.store_scatter(out_v, [rows16, c0], l1 * r2)
            plsc.store_scatter(out_v, [rows16, c1], -(l0 * t))
            plsc.store_scatter(out_v, [rows16, c2], -(l0 * r3))
            plsc.store_scatter(out_v, [rows16, jnp.full((L,), 3, jnp.int32)], -(l0 * r2))
            plsc.store_scatter(out_v, [rows16, jnp.full((L,), 4, jnp.int32)], -(l1 * t))
            plsc.store_scatter(out_v, [rows16, jnp.full((L,), 5, jnp.int32)], -(l1 * r3))
            plsc.store_scatter(out_v, [rows16, jnp.full((L,), 6, jnp.int32)], zero_f)
            plsc.store_scatter(out_v, [rows16, jnp.full((L,), 7, jnp.int32)], s * q)
            plsc.store_scatter(out_v, [rows16, jnp.full((L,), 8, jnp.int32)], -(l2 * r3))

        for g in range(b_per_w // L):
            body(g)

        pltpu.sync_copy(out_v, out_hbm.at[pl.ds(base, b_per_w)])

    return sc_kernel


@jax.jit
def kernel(idx, focal_length, principal_point, T, table):
    batch = idx.shape[0]
    rot_flat = _make_sc_kernel(batch)(table, idx)
    rotmat = rot_flat.reshape(1, batch, 3, 3)
    return (rotmat, focal_length, principal_point, T)


# speedup vs baseline: 40.2644x; 40.2644x over previous
import functools

import jax
import jax.numpy as jnp
from jax import lax
from jax.experimental import pallas as pl
from jax.experimental.pallas import tpu as pltpu
from jax.experimental.pallas import tpu_sc as plsc

NC = 2
NS = 16
L = 16
NW = NC * NS


def _rsqrt(x):
    i = plsc.bitcast(x, jnp.int32)
    i = jnp.int32(0x5F3759DF) - lax.shift_right_logical(i, 1)
    y = plsc.bitcast(i, jnp.float32)
    xh = x * jnp.float32(0.5)
    for _ in range(3):
        y = y * (jnp.float32(1.5) - xh * y * y)
    return y


def _make_sc_kernel(batch):
    b_per_w = batch // NW
    mesh = plsc.VectorSubcoreMesh(
        core_axis_name="c", subcore_axis_name="s", num_cores=NC, num_subcores=NS
    )

    @functools.partial(
        pl.kernel,
        mesh=mesh,
        out_type=jax.ShapeDtypeStruct((batch, 9), jnp.float32),
        scratch_types=[
            pltpu.VMEM((b_per_w,), jnp.int32),
            pltpu.VMEM((b_per_w,), jnp.float32),
            pltpu.VMEM((b_per_w,), jnp.float32),
            pltpu.VMEM((b_per_w,), jnp.float32),
            pltpu.VMEM((b_per_w, 9), jnp.float32),
            pltpu.SemaphoreType.DMA,
            pltpu.SemaphoreType.DMA,
            pltpu.SemaphoreType.DMA,
        ],
        compiler_params=pltpu.CompilerParams(
            needs_layout_passes=False, use_tc_tiling_on_sc=False
        ),
    )
    def sc_kernel(c0_hbm, c1_hbm, c2_hbm, idx_hbm, out_hbm,
                  idx_v, l0_v, l1_v, l2_v, out_v, s0, s1, s2):
        wid = lax.axis_index("s") * NC + lax.axis_index("c")
        base = wid * b_per_w
        pltpu.sync_copy(idx_hbm.at[pl.ds(base, b_per_w)], idx_v)
        cp0 = pltpu.async_copy(c0_hbm.at[idx_v], l0_v, s0)
        cp1 = pltpu.async_copy(c1_hbm.at[idx_v], l1_v, s1)
        cp2 = pltpu.async_copy(c2_hbm.at[idx_v], l2_v, s2)
        cp0.wait()
        cp1.wait()
        cp2.wait()

        lane = lax.iota(jnp.int32, L)
        zero_f = jnp.zeros((L,), jnp.float32)
        cols = [jnp.full((L,), c, jnp.int32) for c in range(9)]

        for g in range(b_per_w // L):
            sl = pl.ds(g * L, L)
            l0 = l0_v[sl]
            l1 = l1_v[sl]
            l2 = l2_v[sl]
            rows16 = lane + g * L

            s = l0 * l0 + l1 * l1
            r2 = _rsqrt(s)
            r3 = _rsqrt(s + l2 * l2)
            q = r2 * r3
            t = l2 * q
            plsc.store_scatter(out_v, [rows16, cols[0]], l1 * r2)
            plsc.store_scatter(out_v, [rows16, cols[1]], -(l0 * t))
            plsc.store_scatter(out_v, [rows16, cols[2]], -(l0 * r3))
            plsc.store_scatter(out_v, [rows16, cols[3]], -(l0 * r2))
            plsc.store_scatter(out_v, [rows16, cols[4]], -(l1 * t))
            plsc.store_scatter(out_v, [rows16, cols[5]], -(l1 * r3))
            plsc.store_scatter(out_v, [rows16, cols[6]], zero_f)
            plsc.store_scatter(out_v, [rows16, cols[7]], s * q)
            plsc.store_scatter(out_v, [rows16, cols[8]], -(l2 * r3))

        pltpu.sync_copy(out_v, out_hbm.at[pl.ds(base, b_per_w)])

    return sc_kernel


@jax.jit
def kernel(idx, focal_length, principal_point, T, table):
    batch = idx.shape[0]
    c0 = table[:, 0]
    c1 = table[:, 1]
    c2 = table[:, 2]
    rot_flat = _make_sc_kernel(batch)(c0, c1, c2, idx)
    rotmat = rot_flat.reshape(1, batch, 3, 3)
    return (rotmat, focal_length, principal_point, T)


# speedup vs baseline: 58.1021x; 1.4430x over previous
import functools

import jax
import jax.numpy as jnp
from jax import lax
from jax.experimental import pallas as pl
from jax.experimental.pallas import tpu as pltpu
from jax.experimental.pallas import tpu_sc as plsc

NC = 2
NS = 16
L = 16
NW = NC * NS


def _rsqrt(x):
    i = plsc.bitcast(x, jnp.int32)
    i = jnp.int32(0x5F3759DF) - lax.shift_right_logical(i, 1)
    y = plsc.bitcast(i, jnp.float32)
    xh = x * jnp.float32(0.5)
    for _ in range(3):
        y = y * (jnp.float32(1.5) - xh * y * y)
    return y


def _make_sc_kernel(batch, n_rows):
    b_per_w = batch // NW
    mesh = plsc.VectorSubcoreMesh(
        core_axis_name="c", subcore_axis_name="s", num_cores=NC, num_subcores=NS
    )

    @functools.partial(
        pl.kernel,
        mesh=mesh,
        out_type=jax.ShapeDtypeStruct((9, batch), jnp.float32),
        scratch_types=[
            pltpu.VMEM((b_per_w,), jnp.int32),
            pltpu.VMEM((b_per_w,), jnp.int32),
            pltpu.VMEM((b_per_w,), jnp.int32),
            pltpu.VMEM((b_per_w,), jnp.float32),
            pltpu.VMEM((b_per_w,), jnp.float32),
            pltpu.VMEM((b_per_w,), jnp.float32),
            pltpu.VMEM((9, b_per_w), jnp.float32),
            pltpu.SemaphoreType.DMA,
            pltpu.SemaphoreType.DMA,
            pltpu.SemaphoreType.DMA,
        ],
        compiler_params=pltpu.CompilerParams(
            needs_layout_passes=False, use_tc_tiling_on_sc=False
        ),
    )
    def sc_kernel(tflat_hbm, idx_hbm, out_hbm,
                  idx0_v, idx1_v, idx2_v, l0_v, l1_v, l2_v, out9_v,
                  s0, s1, s2):
        wid = lax.axis_index("s") * NC + lax.axis_index("c")
        base = wid * b_per_w
        pltpu.sync_copy(idx_hbm.at[pl.ds(base, b_per_w)], idx0_v)
        cp0 = pltpu.async_copy(tflat_hbm.at[idx0_v], l0_v, s0)
        for g in range(b_per_w // L):
            sl = pl.ds(g * L, L)
            i0 = idx0_v[sl]
            idx1_v[sl] = i0 + jnp.int32(n_rows)
            idx2_v[sl] = i0 + jnp.int32(2 * n_rows)
        cp1 = pltpu.async_copy(tflat_hbm.at[idx1_v], l1_v, s1)
        cp2 = pltpu.async_copy(tflat_hbm.at[idx2_v], l2_v, s2)
        cp0.wait()
        cp1.wait()
        cp2.wait()

        zero_f = jnp.zeros((L,), jnp.float32)
        for g in range(b_per_w // L):
            sl = pl.ds(g * L, L)
            l0 = l0_v[sl]
            l1 = l1_v[sl]
            l2 = l2_v[sl]

            s = l0 * l0 + l1 * l1
            r2 = _rsqrt(s)
            r3 = _rsqrt(s + l2 * l2)
            q = r2 * r3
            t = l2 * q
            out9_v[0, sl] = l1 * r2
            out9_v[1, sl] = -(l0 * t)
            out9_v[2, sl] = -(l0 * r3)
            out9_v[3, sl] = -(l0 * r2)
            out9_v[4, sl] = -(l1 * t)
            out9_v[5, sl] = -(l1 * r3)
            out9_v[6, sl] = zero_f
            out9_v[7, sl] = s * q
            out9_v[8, sl] = -(l2 * r3)

        pltpu.sync_copy(out9_v, out_hbm.at[:, pl.ds(base, b_per_w)])

    return sc_kernel


@jax.jit
def kernel(idx, focal_length, principal_point, T, table):
    batch = idx.shape[0]
    n_rows = table.shape[0]
    tflat = table.T.reshape(3 * n_rows)
    out9 = _make_sc_kernel(batch, n_rows)(tflat, idx)
    rotmat = jnp.transpose(out9.reshape(3, 3, 1, batch), (2, 3, 0, 1))
    return (rotmat, focal_length, principal_point, T)
